# baseline (speedup 1.0000x reference)
import functools

import jax
import jax.numpy as jnp
import numpy as np
from jax import lax
from jax.experimental import pallas as pl
from jax.experimental.pallas import tpu as pltpu
from jax.experimental.pallas import tpu_sc as plsc

_PAIRS = np.array(
    [[0, 2], [0, 1], [2, 4], [1, 3], [6, 5], [12, 11], [6, 12], [5, 11],
     [6, 8], [8, 10], [5, 7], [7, 9], [12, 14], [14, 16], [11, 13], [13, 15]],
    dtype=np.int64)

_ROW = 51
_PERIOD = 816

def _build_tables():
    partner = {}
    for v1, v2 in _PAIRS:
        partner[int(v1)] = int(v2)
    delta = np.zeros(_ROW, np.int32)
    mask = np.zeros(_ROW, np.float32)
    for j, p in partner.items():
        delta[j * 3:j * 3 + 3] = (p - j) * 3
        mask[j * 3:j * 3 + 3] = 1.0
    pos = np.arange(_PERIOD)
    tab = (pos % 16) + delta[pos % _ROW]
    return tab.astype(np.int32), mask[pos % _ROW].astype(np.float32)

_TAB_NP, _MASK_NP = _build_tables()

_N = 4096 * 128 * _ROW
_NW = 32
_WPW = _N // _NW
_CHUNK_ROWS = 512
_CW = _CHUNK_ROWS * _ROW
_NCHUNK = _WPW // _CW
_NGROUP = _CW // _PERIOD


def _compute_chunk(in_v, out_v, tab_v, mask_v):
    def group_body(g32, carry):
        base = g32 * _PERIOD
        for g in range(_ROW):
            s = g * 16
            xl = in_v[pl.ds(base + s, 16)]
            idx = tab_v[pl.ds(s, 16)] + (base + s)
            xg = plsc.load_gather(in_v, [idx])
            out_v[pl.ds(base + s, 16)] = (xl - xg) * mask_v[pl.ds(s, 16)]
        return carry
    lax.fori_loop(0, _NGROUP, group_body, 0, unroll=False)


@functools.partial(
    pl.kernel,
    out_type=jax.ShapeDtypeStruct((_N,), jnp.float32),
    mesh=plsc.VectorSubcoreMesh(core_axis_name="c", subcore_axis_name="s"),
    compiler_params=pltpu.CompilerParams(needs_layout_passes=False),
    scratch_types=[
        pltpu.VMEM((_PERIOD,), jnp.int32),
        pltpu.VMEM((_PERIOD,), jnp.float32),
        pltpu.VMEM((_CW,), jnp.float32),
        pltpu.VMEM((_CW,), jnp.float32),
    ],
)
def _sc_joint2bone(x_hbm, tab_hbm, mask_hbm, out_hbm, tab_v, mask_v, in_v,
                   out_v):
    wid = lax.axis_index("s") * 2 + lax.axis_index("c")
    wbase = wid * _WPW
    pltpu.sync_copy(tab_hbm, tab_v)
    pltpu.sync_copy(mask_hbm, mask_v)

    def chunk_body(k, carry):
        off = wbase + k * _CW
        pltpu.sync_copy(x_hbm.at[pl.ds(off, _CW)], in_v)
        _compute_chunk(in_v, out_v, tab_v, mask_v)
        pltpu.sync_copy(out_v, out_hbm.at[pl.ds(off, _CW)])
        return carry

    lax.fori_loop(0, _NCHUNK, chunk_body, 0, unroll=False)


@jax.jit
def kernel(joint):
    x = joint.reshape(-1)
    tab = jnp.asarray(_TAB_NP)
    mask = jnp.asarray(_MASK_NP)
    out = _sc_joint2bone(x, tab, mask)
    return out.reshape(joint.shape)


# speedup vs baseline: 1.0210x; 1.0210x over previous
import functools

import jax
import jax.numpy as jnp
import numpy as np
from jax import lax
from jax.experimental import pallas as pl
from jax.experimental.pallas import tpu as pltpu
from jax.experimental.pallas import tpu_sc as plsc

_PAIRS = np.array(
    [[0, 2], [0, 1], [2, 4], [1, 3], [6, 5], [12, 11], [6, 12], [5, 11],
     [6, 8], [8, 10], [5, 7], [7, 9], [12, 14], [14, 16], [11, 13], [13, 15]],
    dtype=np.int64)

_ROW = 51
_PERIOD = 816

def _build_tables():
    partner = {}
    for v1, v2 in _PAIRS:
        partner[int(v1)] = int(v2)
    delta = np.zeros(_ROW, np.int32)
    mask = np.zeros(_ROW, np.float32)
    for j, p in partner.items():
        delta[j * 3:j * 3 + 3] = (p - j) * 3
        mask[j * 3:j * 3 + 3] = 1.0
    pos = np.arange(_PERIOD)
    tab = (pos % 16) + delta[pos % _ROW]
    return tab.astype(np.int32), mask[pos % _ROW].astype(np.float32)

_TAB_NP, _MASK_NP = _build_tables()

_N = 4096 * 128 * _ROW
_NW = 32
_WPW = _N // _NW
_CHUNK_ROWS = 512
_CW = _CHUNK_ROWS * _ROW
_NCHUNK = _WPW // _CW
_NGROUP = _CW // _PERIOD


def _compute_chunk(in_v, out_v, tab_v):
    def phase_body(g, carry):
        s = g * 16
        tab_reg = tab_v[pl.ds(s, 16)]
        for g32 in range(_NGROUP):
            base = g32 * _PERIOD + s
            xl = in_v[pl.ds(base, 16)]
            xg = plsc.load_gather(in_v, [tab_reg + base])
            out_v[pl.ds(base, 16)] = xl - xg
        return carry
    lax.fori_loop(0, _ROW, phase_body, 0, unroll=False)


@functools.partial(
    pl.kernel,
    out_type=jax.ShapeDtypeStruct((_N,), jnp.float32),
    mesh=plsc.VectorSubcoreMesh(core_axis_name="c", subcore_axis_name="s"),
    compiler_params=pltpu.CompilerParams(needs_layout_passes=False),
    scratch_types=[
        pltpu.VMEM((_PERIOD,), jnp.int32),
        pltpu.VMEM((_CW,), jnp.float32),
        pltpu.VMEM((_CW,), jnp.float32),
        pltpu.VMEM((_CW,), jnp.float32),
        pltpu.VMEM((_CW,), jnp.float32),
        pltpu.SemaphoreType.DMA,
        pltpu.SemaphoreType.DMA,
        pltpu.SemaphoreType.DMA,
        pltpu.SemaphoreType.DMA,
    ],
)
def _sc_joint2bone(x_hbm, tab_hbm, out_hbm, tab_v, in0_v,
                   in1_v, out0_v, out1_v, si0, si1, so0, so1):
    wid = lax.axis_index("s") * 2 + lax.axis_index("c")
    wbase = wid * _WPW
    pltpu.sync_copy(tab_hbm, tab_v)

    ins = (in0_v, in1_v)
    outs = (out0_v, out1_v)
    sis = (si0, si1)
    sos = (so0, so1)

    def start_in(k, b):
        off = wbase + k * _CW
        pltpu.async_copy(x_hbm.at[pl.ds(off, _CW)], ins[b], sis[b])

    def start_out(k, b):
        off = wbase + k * _CW
        pltpu.async_copy(outs[b], out_hbm.at[pl.ds(off, _CW)], sos[b])

    def wait_in(b):
        pltpu.make_async_copy(x_hbm.at[pl.ds(0, _CW)], ins[b], sis[b]).wait()

    def wait_out(b):
        pltpu.make_async_copy(outs[b], out_hbm.at[pl.ds(0, _CW)],
                              sos[b]).wait()

    start_in(0, 0)

    def pair_body(m, carry):
        k = 2 * m
        for b in (0, 1):
            @pl.when(k + b + 1 < _NCHUNK)
            def _():
                start_in(k + b + 1, 1 - b)
            wait_in(b)

            @pl.when(m >= 1)
            def _():
                wait_out(b)
            _compute_chunk(ins[b], outs[b], tab_v)
            start_out(k + b, b)
        return carry

    lax.fori_loop(0, _NCHUNK // 2, pair_body, 0, unroll=False)
    wait_out(0)
    wait_out(1)


@jax.jit
def kernel(joint):
    x = joint.reshape(-1)
    tab = jnp.asarray(_TAB_NP)
    out = _sc_joint2bone(x, tab)
    return out.reshape(joint.shape)


# speedup vs baseline: 96.7242x; 94.7314x over previous
import functools

import jax
import jax.numpy as jnp
from jax import lax
from jax.experimental import pallas as pl
from jax.experimental.pallas import tpu as pltpu
from jax.experimental.pallas import tpu_sc as plsc

_PARTNER = {0: 1, 1: 3, 2: 4, 5: 7, 6: 8, 7: 9, 8: 10, 11: 13, 12: 14,
            13: 15, 14: 16}

_P = 4096 * 128
_NPLANES = 51
_N = _NPLANES * _P
_NW = 32
_SL = _P // _NW
_NV = _SL // 16

_ACTIVE = []
_INACTIVE = []
for _j in range(17):
    for _k in range(3):
        _w = 3 * _j + _k
        if _j in _PARTNER:
            _ACTIVE.append((_w, 3 * _PARTNER[_j] + _k))
        else:
            _INACTIVE.append(_w)


@functools.partial(
    pl.kernel,
    out_type=jax.ShapeDtypeStruct((_N,), jnp.float32),
    mesh=plsc.VectorSubcoreMesh(core_axis_name="c", subcore_axis_name="s"),
    compiler_params=pltpu.CompilerParams(needs_layout_passes=False),
    scratch_types=[
        pltpu.VMEM((_SL,), jnp.float32),
        pltpu.VMEM((_SL,), jnp.float32),
        pltpu.VMEM((_SL,), jnp.float32),
        pltpu.VMEM((_SL,), jnp.float32),
        pltpu.VMEM((_SL,), jnp.float32),
        pltpu.VMEM((_SL,), jnp.float32),
        pltpu.VMEM((_SL,), jnp.float32),
        pltpu.SemaphoreType.DMA,
        pltpu.SemaphoreType.DMA,
        pltpu.SemaphoreType.DMA,
        pltpu.SemaphoreType.DMA,
        pltpu.SemaphoreType.DMA,
        pltpu.SemaphoreType.DMA,
        pltpu.SemaphoreType.DMA,
    ],
)
def _sc_joint2bone(x_hbm, out_hbm, a0, a1, b0, b1, o0, o1, zv,
                   sa0, sa1, sb0, sb1, so0, so1, sz):
    wid = lax.axis_index("s") * 2 + lax.axis_index("c")
    woff = wid * _SL

    zvec = jnp.zeros((16,), jnp.float32)

    def zero_body(i, carry):
        zv[pl.ds(i * 16, 16)] = zvec
        return carry

    lax.fori_loop(0, _NV, zero_body, 0, unroll=8)
    for w in _INACTIVE:
        pltpu.async_copy(zv, out_hbm.at[pl.ds(w * _P + woff, _SL)], sz)

    ins_a = (a0, a1)
    ins_b = (b0, b1)
    outs = (o0, o1)
    sems_a = (sa0, sa1)
    sems_b = (sb0, sb1)
    sems_o = (so0, so1)

    def start_in(k, buf):
        w, w2 = _ACTIVE[k]
        pltpu.async_copy(x_hbm.at[pl.ds(w * _P + woff, _SL)], ins_a[buf],
                         sems_a[buf])
        pltpu.async_copy(x_hbm.at[pl.ds(w2 * _P + woff, _SL)], ins_b[buf],
                         sems_b[buf])

    def wait_in(buf):
        pltpu.make_async_copy(x_hbm.at[pl.ds(0, _SL)], ins_a[buf],
                              sems_a[buf]).wait()
        pltpu.make_async_copy(x_hbm.at[pl.ds(0, _SL)], ins_b[buf],
                              sems_b[buf]).wait()

    def wait_out(buf):
        pltpu.make_async_copy(outs[buf], out_hbm.at[pl.ds(0, _SL)],
                              sems_o[buf]).wait()

    start_in(0, 0)
    for k in range(len(_ACTIVE)):
        buf = k & 1
        if k + 1 < len(_ACTIVE):
            start_in(k + 1, 1 - buf)
        wait_in(buf)
        if k >= 2:
            wait_out(buf)

        xa = ins_a[buf]
        xb = ins_b[buf]
        ov = outs[buf]

        def sub_body(i, carry, xa=xa, xb=xb, ov=ov):
            s = pl.ds(i * 16, 16)
            ov[s] = xa[s] - xb[s]
            return carry

        lax.fori_loop(0, _NV, sub_body, 0, unroll=8)
        pltpu.async_copy(outs[buf],
                         out_hbm.at[pl.ds(_ACTIVE[k][0] * _P + woff, _SL)],
                         sems_o[buf])

    wait_out(0)
    wait_out(1)
    for _ in _INACTIVE:
        pltpu.make_async_copy(zv, out_hbm.at[pl.ds(0, _SL)], sz).wait()


@jax.jit
def kernel(joint):
    x = jnp.transpose(joint, (2, 3, 0, 1)).reshape(-1)
    out = _sc_joint2bone(x)
    return jnp.transpose(out.reshape(17, 3, 4096, 128), (2, 3, 0, 1))


# speedup vs baseline: 193.8791x; 2.0045x over previous
import functools

import jax
import jax.numpy as jnp
from jax import lax
from jax.experimental import pallas as pl
from jax.experimental.pallas import tpu as pltpu
from jax.experimental.pallas import tpu_sc as plsc

_PARTNER = {0: 1, 1: 3, 2: 4, 5: 7, 6: 8, 7: 9, 8: 10, 11: 13, 12: 14,
            13: 15, 14: 16}

_P = 4096 * 128
_NPLANES = 51
_N = _NPLANES * _P
_NW = 32
_WSL = _P // _NW
_NPASS = 2
_SS = _WSL // _NPASS
_NV = _SS // 16
_R = 12
_L = 9

_DELTA = {}
_INACTIVE = []
for _j in range(17):
    for _k in range(3):
        _w = 3 * _j + _k
        if _j in _PARTNER:
            _DELTA[_w] = 3 * _PARTNER[_j] + _k
        else:
            _INACTIVE.append(_w)


@functools.partial(
    pl.kernel,
    out_type=jax.ShapeDtypeStruct((_N,), jnp.float32),
    mesh=plsc.VectorSubcoreMesh(core_axis_name="c", subcore_axis_name="s"),
    compiler_params=pltpu.CompilerParams(needs_layout_passes=False),
    scratch_types=(
        [pltpu.VMEM((_SS,), jnp.float32) for _ in range(_R)]
        + [pltpu.VMEM((_SS,), jnp.float32)]
        + [pltpu.SemaphoreType.DMA for _ in range(2 * _R + 1)]
    ),
)
def _sc_joint2bone(x_hbm, out_hbm, *refs):
    ring = refs[:_R]
    zv = refs[_R]
    lsem = refs[_R + 1:2 * _R + 1]
    ssem = refs[2 * _R + 1:3 * _R + 1]
    zsem = refs[3 * _R + 1]

    wid = lax.axis_index("s") * 2 + lax.axis_index("c")
    wbase = wid * _WSL

    zvec = jnp.zeros((16,), jnp.float32)

    @plsc.parallel_loop(0, _NV, 1, unroll=8)
    def zero_body(i):
        zv[pl.ds(i * 16, 16)] = zvec

    def start_load(w, off):
        pltpu.async_copy(x_hbm.at[pl.ds(w * _P + off, _SS)], ring[w % _R],
                         lsem[w % _R])

    def wait_load(w):
        pltpu.make_async_copy(x_hbm.at[pl.ds(0, _SS)], ring[w % _R],
                              lsem[w % _R]).wait()

    def start_store(w, off):
        pltpu.async_copy(ring[w % _R], out_hbm.at[pl.ds(w * _P + off, _SS)],
                         ssem[w % _R])

    def wait_store(w):
        pltpu.make_async_copy(ring[w % _R], out_hbm.at[pl.ds(0, _SS)],
                              ssem[w % _R]).wait()

    for c in range(_NPASS):
        off = wbase + c * _SS
        for w in _INACTIVE:
            pltpu.async_copy(zv, out_hbm.at[pl.ds(w * _P + off, _SS)], zsem)

        loads_waited = set()
        stores_unwaited = set()
        for w in range(_L):
            start_load(w, off)
        for w in range(_NPLANES):
            nxt = w + _L
            if nxt < _NPLANES:
                prev = nxt - _R
                if prev in stores_unwaited:
                    wait_store(prev)
                    stores_unwaited.discard(prev)
                start_load(nxt, off)
            if w in _DELTA:
                for q in (w, _DELTA[w]):
                    if q not in loads_waited:
                        wait_load(q)
                        loads_waited.add(q)
                xa = ring[w % _R]
                xb = ring[_DELTA[w] % _R]

                @plsc.parallel_loop(0, _NV, 1, unroll=8)
                def sub_body(i, xa=xa, xb=xb):
                    s = pl.ds(i * 16, 16)
                    xa[s] = xa[s] - xb[s]
                start_store(w, off)
                stores_unwaited.add(w)
        for w in sorted(stores_unwaited):
            wait_store(w)
        for _ in _INACTIVE:
            pltpu.make_async_copy(zv, out_hbm.at[pl.ds(0, _SS)], zsem).wait()


@jax.jit
def kernel(joint):
    x = jnp.transpose(joint, (2, 3, 0, 1)).reshape(-1)
    out = _sc_joint2bone(x)
    return jnp.transpose(out.reshape(17, 3, 4096, 128), (2, 3, 0, 1))


# speedup vs baseline: 201.7558x; 1.0406x over previous
import functools

import jax
import jax.numpy as jnp
from jax import lax
from jax.experimental import pallas as pl
from jax.experimental.pallas import tpu as pltpu
from jax.experimental.pallas import tpu_sc as plsc

_PARTNER = {0: 1, 1: 3, 2: 4, 5: 7, 6: 8, 7: 9, 8: 10, 11: 13, 12: 14,
            13: 15, 14: 16}

_P = 4096 * 128
_NPLANES = 51
_N = _NPLANES * _P
_NW = 32
_SS = _P // _NW
_NV = _SS // 16
_R = 6
_L = 4

_DELTA = {}
_INACTIVE = []
for _j in range(17):
    for _k in range(3):
        _w = 3 * _j + _k
        if _j in _PARTNER:
            _DELTA[_w] = 3 * _PARTNER[_j] + _k
        else:
            _INACTIVE.append(_w)

_P_ORDER = [3 * _i + _k for _k in range(3) for _i in range(17)]
_POS = {_w: _g for _g, _w in enumerate(_P_ORDER)}


@functools.partial(
    pl.kernel,
    out_type=jax.ShapeDtypeStruct((_N,), jnp.float32),
    mesh=plsc.VectorSubcoreMesh(core_axis_name="c", subcore_axis_name="s"),
    compiler_params=pltpu.CompilerParams(needs_layout_passes=False),
    scratch_types=(
        [pltpu.VMEM((_SS,), jnp.float32) for _ in range(_R + 1)]
        + [pltpu.SemaphoreType.DMA for _ in range(2 * _R + 1)]
    ),
)
def _sc_joint2bone(x_hbm, out_hbm, *refs):
    ring = refs[:_R]
    zv = refs[_R]
    lsem = refs[_R + 1:2 * _R + 1]
    ssem = refs[2 * _R + 1:3 * _R + 1]
    zsem = refs[3 * _R + 1]

    wid = lax.axis_index("s") * 2 + lax.axis_index("c")
    woff = wid * _SS

    zvec = jnp.zeros((16,), jnp.float32)

    @plsc.parallel_loop(0, _NV, 1, unroll=8)
    def zero_body(i):
        zv[pl.ds(i * 16, 16)] = zvec

    def slot(w):
        return _POS[w] % _R

    def start_load(w):
        pltpu.async_copy(x_hbm.at[pl.ds(w * _P + woff, _SS)], ring[slot(w)],
                         lsem[slot(w)])

    def wait_load(w):
        pltpu.make_async_copy(x_hbm.at[pl.ds(0, _SS)], ring[slot(w)],
                              lsem[slot(w)]).wait()

    def start_store(w):
        pltpu.async_copy(ring[slot(w)],
                         out_hbm.at[pl.ds(w * _P + woff, _SS)], ssem[slot(w)])

    def wait_store(w):
        pltpu.make_async_copy(ring[slot(w)], out_hbm.at[pl.ds(0, _SS)],
                              ssem[slot(w)]).wait()

    loads_waited = set()
    stores_unwaited = set()
    for g in range(_L):
        start_load(_P_ORDER[g])
    for g in range(_NPLANES):
        w = _P_ORDER[g]
        nxt = g + _L
        if nxt < _NPLANES:
            prev = nxt - _R
            if prev >= 0 and _P_ORDER[prev] in stores_unwaited:
                wait_store(_P_ORDER[prev])
                stores_unwaited.discard(_P_ORDER[prev])
            start_load(_P_ORDER[nxt])
        if w in _DELTA:
            for q in (w, _DELTA[w]):
                if q not in loads_waited:
                    wait_load(q)
                    loads_waited.add(q)
            xa = ring[slot(w)]
            xb = ring[slot(_DELTA[w])]

            @plsc.parallel_loop(0, _NV, 1, unroll=8)
            def sub_body(i, xa=xa, xb=xb):
                s = pl.ds(i * 16, 16)
                xa[s] = xa[s] - xb[s]

            start_store(w)
            stores_unwaited.add(w)
        else:
            pltpu.async_copy(zv, out_hbm.at[pl.ds(w * _P + woff, _SS)], zsem)
    for w in [p for p in _P_ORDER if p in stores_unwaited]:
        wait_store(w)
    for _ in _INACTIVE:
        pltpu.make_async_copy(zv, out_hbm.at[pl.ds(0, _SS)], zsem).wait()


@jax.jit
def kernel(joint):
    x = jnp.transpose(joint, (2, 3, 0, 1)).reshape(-1)
    out = _sc_joint2bone(x)
    return jnp.transpose(out.reshape(17, 3, 4096, 128), (2, 3, 0, 1))


# speedup vs baseline: 205.0130x; 1.0161x over previous
import functools

import jax
import jax.numpy as jnp
from jax import lax
from jax.experimental import pallas as pl
from jax.experimental.pallas import tpu as pltpu
from jax.experimental.pallas import tpu_sc as plsc

_PARTNER = {0: 1, 1: 3, 2: 4, 5: 7, 6: 8, 7: 9, 8: 10, 11: 13, 12: 14,
            13: 15, 14: 16}

_P = 4096 * 128
_NPLANES = 51
_N = _NPLANES * _P
_NW = 32
_SS = _P // _NW
_NV = _SS // 16
_R = 7
_L = 5
_ZS = _SS // 2

_DELTA = {}
_INACTIVE = []
for _j in range(17):
    for _k in range(3):
        _w = 3 * _j + _k
        if _j in _PARTNER:
            _DELTA[_w] = 3 * _PARTNER[_j] + _k
        else:
            _INACTIVE.append(_w)

_P_ORDER = [3 * _i + _k for _k in range(3) for _i in range(17)]
_POS = {_w: _g for _g, _w in enumerate(_P_ORDER)}


@functools.partial(
    pl.kernel,
    out_type=jax.ShapeDtypeStruct((_N,), jnp.float32),
    mesh=plsc.VectorSubcoreMesh(core_axis_name="c", subcore_axis_name="s"),
    compiler_params=pltpu.CompilerParams(needs_layout_passes=False),
    scratch_types=(
        [pltpu.VMEM((_SS,), jnp.float32) for _ in range(_R)]
        + [pltpu.VMEM((_ZS,), jnp.float32)]
        + [pltpu.SemaphoreType.DMA for _ in range(2 * _R + 1)]
    ),
)
def _sc_joint2bone(x_hbm, out_hbm, *refs):
    ring = refs[:_R]
    zv = refs[_R]
    lsem = refs[_R + 1:2 * _R + 1]
    ssem = refs[2 * _R + 1:3 * _R + 1]
    zsem = refs[3 * _R + 1]

    wid = lax.axis_index("s") * 2 + lax.axis_index("c")
    woff = wid * _SS

    zvec = jnp.zeros((16,), jnp.float32)

    @plsc.parallel_loop(0, _ZS // 16, 1, unroll=8)
    def zero_body(i):
        zv[pl.ds(i * 16, 16)] = zvec

    def slot(w):
        return _POS[w] % _R

    def start_load(w):
        pltpu.async_copy(x_hbm.at[pl.ds(w * _P + woff, _SS)], ring[slot(w)],
                         lsem[slot(w)])

    def wait_load(w):
        pltpu.make_async_copy(x_hbm.at[pl.ds(0, _SS)], ring[slot(w)],
                              lsem[slot(w)]).wait()

    def start_store(w):
        pltpu.async_copy(ring[slot(w)],
                         out_hbm.at[pl.ds(w * _P + woff, _SS)], ssem[slot(w)])

    def wait_store(w):
        pltpu.make_async_copy(ring[slot(w)], out_hbm.at[pl.ds(0, _SS)],
                              ssem[slot(w)]).wait()

    loads_waited = set()
    stores_unwaited = set()
    for g in range(_L):
        start_load(_P_ORDER[g])
    for g in range(_NPLANES):
        w = _P_ORDER[g]
        nxt = g + _L
        if nxt < _NPLANES:
            prev = nxt - _R
            if prev >= 0 and _P_ORDER[prev] in stores_unwaited:
                wait_store(_P_ORDER[prev])
                stores_unwaited.discard(_P_ORDER[prev])
            start_load(_P_ORDER[nxt])
        if w in _DELTA:
            for q in (w, _DELTA[w]):
                if q not in loads_waited:
                    wait_load(q)
                    loads_waited.add(q)
            xa = ring[slot(w)]
            xb = ring[slot(_DELTA[w])]

            @plsc.parallel_loop(0, _NV, 1, unroll=8)
            def sub_body(i, xa=xa, xb=xb):
                s = pl.ds(i * 16, 16)
                xa[s] = xa[s] - xb[s]

            start_store(w)
            stores_unwaited.add(w)
        else:
            for h in range(2):
                pltpu.async_copy(
                    zv, out_hbm.at[pl.ds(w * _P + woff + h * _ZS, _ZS)],
                    zsem)
    for w in [p for p in _P_ORDER if p in stores_unwaited]:
        wait_store(w)
    for _ in range(2 * len(_INACTIVE)):
        pltpu.make_async_copy(zv, out_hbm.at[pl.ds(0, _ZS)], zsem).wait()


@jax.jit
def kernel(joint):
    x = jnp.transpose(joint, (2, 3, 0, 1)).reshape(-1)
    out = _sc_joint2bone(x)
    return jnp.transpose(out.reshape(17, 3, 4096, 128), (2, 3, 0, 1))
